# baseline (speedup 1.0000x reference)
import functools

import jax
import jax.numpy as jnp
from jax import lax
from jax.experimental import pallas as pl
from jax.experimental.pallas import tpu as pltpu
from jax.experimental.pallas import tpu_sc as plsc

_NC = 2
_NS = 16
_NW = _NC * _NS
_CHUNK = 64


def _sc_mesh():
    return plsc.VectorSubcoreMesh(
        core_axis_name="c", subcore_axis_name="s", num_cores=_NC, num_subcores=_NS
    )


def _sc_degree(dst3, zeros_hbm, npad, nch):
    rps = npad // _NS

    @functools.partial(
        pl.kernel,
        out_type=jax.ShapeDtypeStruct((_NC, npad), jnp.float32),
        mesh=_sc_mesh(),
        scratch_types=[
            pltpu.VMEM((nch, _CHUNK), jnp.int32),
            pltpu.VMEM((_CHUNK,), jnp.float32),
            pltpu.VMEM_SHARED((npad,), jnp.float32),
        ],
    )
    def k(dst_hbm, zeros_h, ones_h, out_hbm, didx, ones_v, acc):
        c = lax.axis_index("c")
        s = lax.axis_index("s")
        wid = s * _NC + c
        r0 = s * rps
        pltpu.sync_copy(zeros_h.at[pl.ds(r0, rps)], acc.at[pl.ds(r0, rps)])
        pltpu.sync_copy(ones_h, ones_v)
        pltpu.sync_copy(dst_hbm.at[wid], didx)
        plsc.subcore_barrier()

        def body(i, carry):
            pltpu.sync_copy(ones_v, acc.at[didx.at[i]], add=True)
            return carry

        lax.fori_loop(0, nch, body, 0)
        plsc.subcore_barrier()
        pltpu.sync_copy(acc.at[pl.ds(r0, rps)], out_hbm.at[c, pl.ds(r0, rps)])

    return k(dst3, zeros_hbm, jnp.ones((_CHUNK,), jnp.float32))


_NBUF = 4
_LOOK = 2
_IDXB = 2 * _NBUF


def _sc_edge_agg(h, ei4, zeros_hbm, npad, nch):
    n, d = h.shape
    rps = npad // _NS
    assert nch % _IDXB == 0

    @functools.partial(
        pl.kernel,
        out_type=jax.ShapeDtypeStruct((_NC, npad, d), jnp.float32),
        mesh=_sc_mesh(),
        scratch_types=[pltpu.VMEM((_CHUNK, d), jnp.float32)] * _NBUF
        + [pltpu.VMEM((2, _CHUNK), jnp.int32)] * _IDXB
        + [pltpu.VMEM_SHARED((npad, d), jnp.float32)]
        + [pltpu.SemaphoreType.DMA] * (2 * _NBUF + _IDXB),
    )
    def k(h_hbm, ei_hbm, zeros_h, out_hbm, *rest):
        rows = rest[:_NBUF]
        idxs = rest[_NBUF : _NBUF + _IDXB]
        acc = rest[_NBUF + _IDXB]
        sems = rest[_NBUF + _IDXB + 1 :]
        gsem = sems[:_NBUF]
        ssem = sems[_NBUF : 2 * _NBUF]
        isem = sems[2 * _NBUF :]
        c = lax.axis_index("c")
        s = lax.axis_index("s")
        wid = s * _NC + c
        r0 = s * rps
        pltpu.sync_copy(zeros_h, acc.at[pl.ds(r0, rps)])

        def idx_start(chunk, q):
            pltpu.async_copy(ei_hbm.at[wid, chunk], idxs[q], isem[q])

        def idx_wait(q):
            pltpu.make_async_copy(ei_hbm.at[0, 0], idxs[q], isem[q]).wait()

        def gather_start(q, b):
            pltpu.async_copy(h_hbm.at[idxs[q].at[0]], rows[b], gsem[b])

        def gather_wait(b):
            pltpu.make_async_copy(h_hbm.at[pl.ds(0, _CHUNK)], rows[b], gsem[b]).wait()

        def scatter_start(q, b):
            pltpu.async_copy(rows[b], acc.at[idxs[q].at[1]], ssem[b], add=True)

        def scatter_wait(b):
            pltpu.make_async_copy(rows[b], acc.at[pl.ds(0, _CHUNK)], ssem[b]).wait()

        for j in range(_NBUF):
            idx_start(j, j)
        plsc.subcore_barrier()

        def emit_iter(i, u, first):
            b = u % _NBUF
            q = u % _IDXB
            idx_wait(q)
            if not first or u >= _NBUF:
                scatter_wait(b)
            gather_start(q, b)
            idx_start(lax.rem(i + _NBUF, nch), (u + _NBUF) % _IDXB)
            if not first or u >= _LOOK:
                bl = (u - _LOOK) % _NBUF
                gather_wait(bl)
                scatter_start((u - _LOOK) % _IDXB, bl)

        for u in range(_IDXB):
            emit_iter(jnp.int32(u), u, True)

        def blk(t, carry):
            for u in range(_IDXB):
                emit_iter(t * _IDXB + u, u, False)
            return carry

        lax.fori_loop(1, nch // _IDXB, blk, 0)
        for j in range(_LOOK):
            u = nch + j
            bl = (u - _LOOK) % _NBUF
            gather_wait(bl)
            scatter_start((u - _LOOK) % _IDXB, bl)
        for b in range(_NBUF):
            scatter_wait(b)
        for j in range(_NBUF):
            idx_wait((nch + j) % _IDXB)
        plsc.subcore_barrier()
        pltpu.sync_copy(acc.at[pl.ds(r0, rps)], out_hbm.at[c, pl.ds(r0, rps)])

    return k(h, ei4, zeros_hbm)


def _tc_layer1(x, w1, d0, d1, bm=1000):
    m, d = x.shape

    def body(x_ref, w_ref, d0_ref, d1_ref, o_ref):
        dis = lax.rsqrt(d0_ref[...] + d1_ref[...] + 1.0)
        o_ref[...] = (
            jnp.dot(x_ref[...], w_ref[...], preferred_element_type=jnp.float32) * dis
        )

    return pl.pallas_call(
        body,
        grid=(m // bm,),
        in_specs=[
            pl.BlockSpec((bm, d), lambda i: (i, 0)),
            pl.BlockSpec((d, d), lambda i: (0, 0)),
            pl.BlockSpec((bm, 1), lambda i: (i, 0)),
            pl.BlockSpec((bm, 1), lambda i: (i, 0)),
        ],
        out_specs=pl.BlockSpec((bm, d), lambda i: (i, 0)),
        out_shape=jax.ShapeDtypeStruct((m, d), jnp.float32),
    )(x, w1, d0, d1)


def _tc_layer2(p0, p1, h1p, d0, d1, b1, w2, bm=1000):
    m, d = h1p.shape

    def body(p0_ref, p1_ref, h_ref, d0_ref, d1_ref, b_ref, w_ref, o_ref):
        dis = lax.rsqrt(d0_ref[...] + d1_ref[...] + 1.0)
        z = dis * (p0_ref[...] + p1_ref[...] + h_ref[...]) + b_ref[...]
        z = jnp.maximum(z, 0.0)
        o_ref[...] = (
            jnp.dot(z, w_ref[...], preferred_element_type=jnp.float32) * dis
        )

    row = pl.BlockSpec((bm, d), lambda i: (i, 0))
    return pl.pallas_call(
        body,
        grid=(m // bm,),
        in_specs=[
            row,
            row,
            row,
            pl.BlockSpec((bm, 1), lambda i: (i, 0)),
            pl.BlockSpec((bm, 1), lambda i: (i, 0)),
            pl.BlockSpec((1, d), lambda i: (0, 0)),
            pl.BlockSpec((d, d), lambda i: (0, 0)),
        ],
        out_specs=row,
        out_shape=jax.ShapeDtypeStruct((m, d), jnp.float32),
    )(p0, p1, h1p, d0, d1, b1, w2)


def _tc_layer3(q0, q1, h2p, d0, d1, b2, bm=1000):
    m, d = h2p.shape

    def body(q0_ref, q1_ref, h_ref, d0_ref, d1_ref, b_ref, o_ref):
        dis = lax.rsqrt(d0_ref[...] + d1_ref[...] + 1.0)
        o_ref[...] = dis * (q0_ref[...] + q1_ref[...] + h_ref[...]) + b_ref[...]

    row = pl.BlockSpec((bm, d), lambda i: (i, 0))
    return pl.pallas_call(
        body,
        grid=(m // bm,),
        in_specs=[
            row,
            row,
            row,
            pl.BlockSpec((bm, 1), lambda i: (i, 0)),
            pl.BlockSpec((bm, 1), lambda i: (i, 0)),
            pl.BlockSpec((1, d), lambda i: (0, 0)),
        ],
        out_specs=row,
        out_shape=jax.ShapeDtypeStruct((m, d), jnp.float32),
    )(q0, q1, h2p, d0, d1, b2)


def kernel(x, edge_index, W1, b1, W2, b2):
    n, d = x.shape
    e = edge_index.shape[1]

    nch = _IDXB * (-(-e // (_NW * _CHUNK * _IDXB)))
    epw = nch * _CHUNK
    ep = epw * _NW
    npad = 256 * (-(-(n + 1) // 256))

    src = edge_index[0].astype(jnp.int32)
    dst = edge_index[1].astype(jnp.int32)
    pad = ep - e
    src3 = jnp.concatenate([src, jnp.zeros((pad,), jnp.int32)]).reshape(
        _NW, nch, _CHUNK
    )
    dst3 = jnp.concatenate(
        [dst, jnp.full((pad,), n, jnp.int32)]
    ).reshape(_NW, nch, _CHUNK)
    ei4 = jnp.stack([src3, dst3], axis=2)

    rps = npad // _NS
    zeros_1d = jnp.zeros((npad,), jnp.float32)
    zeros_d = jnp.zeros((rps, d), jnp.float32)

    degp = _sc_degree(dst3, zeros_1d, npad, nch)
    d0 = degp[0, :n].reshape(n, 1)
    d1 = degp[1, :n].reshape(n, 1)

    h1p = _tc_layer1(x, W1, d0, d1)

    p = _sc_edge_agg(h1p, ei4, zeros_d, npad, nch)
    h2p = _tc_layer2(p[0, :n, :], p[1, :n, :], h1p, d0, d1, b1.reshape(1, d), W2)

    q = _sc_edge_agg(h2p, ei4, zeros_d, npad, nch)
    return _tc_layer3(q[0, :n, :], q[1, :n, :], h2p, d0, d1, b2.reshape(1, d))


# speedup vs baseline: 2.9747x; 2.9747x over previous
import functools

import jax
import jax.numpy as jnp
from jax import lax
from jax.experimental import pallas as pl
from jax.experimental.pallas import tpu as pltpu
from jax.experimental.pallas import tpu_sc as plsc

_NC = 2
_NS = 16
_NW = _NC * _NS
_CHUNK = 64


def _sc_mesh():
    return plsc.VectorSubcoreMesh(
        core_axis_name="c", subcore_axis_name="s", num_cores=_NC, num_subcores=_NS
    )


def _sc_degree(dst3, zeros_hbm, npad, nch):
    rps = npad // _NS

    @functools.partial(
        pl.kernel,
        out_type=jax.ShapeDtypeStruct((_NC, npad), jnp.float32),
        mesh=_sc_mesh(),
        scratch_types=[
            pltpu.VMEM((nch, _CHUNK), jnp.int32),
            pltpu.VMEM((_CHUNK,), jnp.float32),
            pltpu.VMEM_SHARED((npad,), jnp.float32),
        ],
    )
    def k(dst_hbm, zeros_h, ones_h, out_hbm, didx, ones_v, acc):
        c = lax.axis_index("c")
        s = lax.axis_index("s")
        wid = s * _NC + c
        r0 = s * rps
        pltpu.sync_copy(zeros_h.at[pl.ds(r0, rps)], acc.at[pl.ds(r0, rps)])
        pltpu.sync_copy(ones_h, ones_v)
        pltpu.sync_copy(dst_hbm.at[wid], didx)
        plsc.subcore_barrier()

        def body(i, carry):
            pltpu.sync_copy(ones_v, acc.at[didx.at[i]], add=True)
            return carry

        lax.fori_loop(0, nch, body, 0)
        plsc.subcore_barrier()
        pltpu.sync_copy(acc.at[pl.ds(r0, rps)], out_hbm.at[c, pl.ds(r0, rps)])

    return k(dst3, zeros_hbm, jnp.ones((_CHUNK,), jnp.float32))


_NBUF = 4
_LOOK = 2
_IDXB = 2 * _NBUF


def _sc_edge_agg(h, ei4, zeros_hbm, npad, nch):
    n, d = h.shape
    rps = npad // _NS
    assert nch % _IDXB == 0

    @functools.partial(
        pl.kernel,
        out_type=jax.ShapeDtypeStruct((_NC, npad, d), jnp.float32),
        mesh=_sc_mesh(),
        scratch_types=[pltpu.VMEM((_CHUNK, d), jnp.float32)] * _NBUF
        + [pltpu.VMEM((2, _CHUNK), jnp.int32)] * _IDXB
        + [pltpu.VMEM_SHARED((npad, d), jnp.float32)]
        + [pltpu.SemaphoreType.DMA] * (2 * _NBUF + _IDXB),
    )
    def k(h_hbm, ei_hbm, zeros_h, out_hbm, *rest):
        rows = rest[:_NBUF]
        idxs = rest[_NBUF : _NBUF + _IDXB]
        acc = rest[_NBUF + _IDXB]
        sems = rest[_NBUF + _IDXB + 1 :]
        gsem = sems[:_NBUF]
        ssem = sems[_NBUF : 2 * _NBUF]
        isem = sems[2 * _NBUF :]
        c = lax.axis_index("c")
        s = lax.axis_index("s")
        wid = s * _NC + c
        r0 = s * rps
        pltpu.sync_copy(zeros_h, acc.at[pl.ds(r0, rps)])

        def idx_start(chunk, q):
            pltpu.async_copy(ei_hbm.at[wid, chunk], idxs[q], isem[q])

        def idx_wait(q):
            pltpu.make_async_copy(ei_hbm.at[0, 0], idxs[q], isem[q]).wait()

        def gather_start(q, b):
            pltpu.async_copy(h_hbm.at[idxs[q].at[0]], rows[b], gsem[b])

        def gather_wait(b):
            pltpu.make_async_copy(h_hbm.at[pl.ds(0, _CHUNK)], rows[b], gsem[b]).wait()

        def scatter_start(q, b):
            pltpu.async_copy(rows[b], acc.at[idxs[q].at[1]], ssem[b], add=True)

        def scatter_wait(b):
            pltpu.make_async_copy(rows[b], acc.at[pl.ds(0, _CHUNK)], ssem[b]).wait()

        for j in range(_NBUF):
            idx_start(j, j)
        plsc.subcore_barrier()

        def emit_iter(i, u, first):
            b = u % _NBUF
            q = u % _IDXB
            idx_wait(q)
            if not first or u >= _NBUF:
                scatter_wait(b)
            gather_start(q, b)
            idx_start(lax.rem(i + _NBUF, nch), (u + _NBUF) % _IDXB)
            if not first or u >= _LOOK:
                bl = (u - _LOOK) % _NBUF
                gather_wait(bl)
                scatter_start((u - _LOOK) % _IDXB, bl)

        for u in range(_IDXB):
            emit_iter(jnp.int32(u), u, True)

        def blk(t, carry):
            for u in range(_IDXB):
                emit_iter(t * _IDXB + u, u, False)
            return carry

        lax.fori_loop(1, nch // _IDXB, blk, 0)
        for j in range(_LOOK):
            u = nch + j
            bl = (u - _LOOK) % _NBUF
            gather_wait(bl)
            scatter_start((u - _LOOK) % _IDXB, bl)
        for b in range(_NBUF):
            scatter_wait(b)
        for j in range(_NBUF):
            idx_wait((nch + j) % _IDXB)
        plsc.subcore_barrier()
        pltpu.sync_copy(acc.at[pl.ds(r0, rps)], out_hbm.at[c, pl.ds(r0, rps)])

    return k(h, ei4, zeros_hbm)


def _tc_layer1(x, w1, d0, d1, bm=1000):
    m, d = x.shape

    def body(x_ref, w_ref, d0_ref, d1_ref, o_ref):
        dis = lax.rsqrt(d0_ref[...] + d1_ref[...] + 1.0)
        o_ref[...] = (
            jnp.dot(x_ref[...], w_ref[...], preferred_element_type=jnp.float32) * dis
        )

    return pl.pallas_call(
        body,
        grid=(m // bm,),
        in_specs=[
            pl.BlockSpec((bm, d), lambda i: (i, 0)),
            pl.BlockSpec((d, d), lambda i: (0, 0)),
            pl.BlockSpec((bm, 1), lambda i: (i, 0)),
            pl.BlockSpec((bm, 1), lambda i: (i, 0)),
        ],
        out_specs=pl.BlockSpec((bm, d), lambda i: (i, 0)),
        out_shape=jax.ShapeDtypeStruct((m, d), jnp.float32),
    )(x, w1, d0, d1)


def _tc_layer2(p0, p1, h1p, d0, d1, b1, w2, bm=1000):
    m, d = h1p.shape

    def body(p0_ref, p1_ref, h_ref, d0_ref, d1_ref, b_ref, w_ref, o_ref):
        dis = lax.rsqrt(d0_ref[...] + d1_ref[...] + 1.0)
        z = dis * (p0_ref[...] + p1_ref[...] + h_ref[...]) + b_ref[...]
        z = jnp.maximum(z, 0.0)
        o_ref[...] = (
            jnp.dot(z, w_ref[...], preferred_element_type=jnp.float32) * dis
        )

    row = pl.BlockSpec((bm, d), lambda i: (i, 0))
    return pl.pallas_call(
        body,
        grid=(m // bm,),
        in_specs=[
            row,
            row,
            row,
            pl.BlockSpec((bm, 1), lambda i: (i, 0)),
            pl.BlockSpec((bm, 1), lambda i: (i, 0)),
            pl.BlockSpec((1, d), lambda i: (0, 0)),
            pl.BlockSpec((d, d), lambda i: (0, 0)),
        ],
        out_specs=row,
        out_shape=jax.ShapeDtypeStruct((m, d), jnp.float32),
    )(p0, p1, h1p, d0, d1, b1, w2)


def _tc_layer3(q0, q1, h2p, d0, d1, b2, bm=1000):
    m, d = h2p.shape

    def body(q0_ref, q1_ref, h_ref, d0_ref, d1_ref, b_ref, o_ref):
        dis = lax.rsqrt(d0_ref[...] + d1_ref[...] + 1.0)
        o_ref[...] = dis * (q0_ref[...] + q1_ref[...] + h_ref[...]) + b_ref[...]

    row = pl.BlockSpec((bm, d), lambda i: (i, 0))
    return pl.pallas_call(
        body,
        grid=(m // bm,),
        in_specs=[
            row,
            row,
            row,
            pl.BlockSpec((bm, 1), lambda i: (i, 0)),
            pl.BlockSpec((bm, 1), lambda i: (i, 0)),
            pl.BlockSpec((1, d), lambda i: (0, 0)),
        ],
        out_specs=row,
        out_shape=jax.ShapeDtypeStruct((m, d), jnp.float32),
    )(q0, q1, h2p, d0, d1, b2)


def kernel(x, edge_index, W1, b1, W2, b2):
    n, d = x.shape
    e = edge_index.shape[1]

    nch = _IDXB * (-(-e // (_NW * _CHUNK * _IDXB)))
    epw = nch * _CHUNK
    ep = epw * _NW
    npad = 256 * (-(-(n + 1) // 256))

    src = edge_index[0].astype(jnp.int32)
    dst = edge_index[1].astype(jnp.int32)
    pad = ep - e
    pad_src = jnp.arange(pad, dtype=jnp.int32) % n
    pad_dst = n + jnp.arange(pad, dtype=jnp.int32) % (npad - n)
    src3 = jnp.concatenate([src, pad_src]).reshape(_NW, nch, _CHUNK)
    dst3 = jnp.concatenate([dst, pad_dst]).reshape(_NW, nch, _CHUNK)
    ei4 = jnp.stack([src3, dst3], axis=2)

    rps = npad // _NS
    zeros_1d = jnp.zeros((npad,), jnp.float32)
    zeros_d = jnp.zeros((rps, d), jnp.float32)

    degp = _sc_degree(dst3, zeros_1d, npad, nch)
    d0 = degp[0, :n].reshape(n, 1)
    d1 = degp[1, :n].reshape(n, 1)

    h1p = _tc_layer1(x, W1, d0, d1)

    p = _sc_edge_agg(h1p, ei4, zeros_d, npad, nch)
    h2p = _tc_layer2(p[0, :n, :], p[1, :n, :], h1p, d0, d1, b1.reshape(1, d), W2)

    q = _sc_edge_agg(h2p, ei4, zeros_d, npad, nch)
    return _tc_layer3(q[0, :n, :], q[1, :n, :], h2p, d0, d1, b2.reshape(1, d))


# speedup vs baseline: 2.9825x; 1.0026x over previous
import functools

import jax
import jax.numpy as jnp
from jax import lax
from jax.experimental import pallas as pl
from jax.experimental.pallas import tpu as pltpu
from jax.experimental.pallas import tpu_sc as plsc

_NC = 2
_NS = 16
_NW = _NC * _NS
_CHUNK = 128


def _sc_mesh():
    return plsc.VectorSubcoreMesh(
        core_axis_name="c", subcore_axis_name="s", num_cores=_NC, num_subcores=_NS
    )


def _sc_degree(dst3, zeros_hbm, npad, nch):
    rps = npad // _NS

    @functools.partial(
        pl.kernel,
        out_type=jax.ShapeDtypeStruct((_NC, npad), jnp.float32),
        mesh=_sc_mesh(),
        scratch_types=[
            pltpu.VMEM((nch, _CHUNK), jnp.int32),
            pltpu.VMEM((_CHUNK,), jnp.float32),
            pltpu.VMEM_SHARED((npad,), jnp.float32),
        ],
    )
    def k(dst_hbm, zeros_h, ones_h, out_hbm, didx, ones_v, acc):
        c = lax.axis_index("c")
        s = lax.axis_index("s")
        wid = s * _NC + c
        r0 = s * rps
        pltpu.sync_copy(zeros_h.at[pl.ds(r0, rps)], acc.at[pl.ds(r0, rps)])
        pltpu.sync_copy(ones_h, ones_v)
        pltpu.sync_copy(dst_hbm.at[wid], didx)
        plsc.subcore_barrier()

        def body(i, carry):
            pltpu.sync_copy(ones_v, acc.at[didx.at[i]], add=True)
            return carry

        lax.fori_loop(0, nch, body, 0)
        plsc.subcore_barrier()
        pltpu.sync_copy(acc.at[pl.ds(r0, rps)], out_hbm.at[c, pl.ds(r0, rps)])

    return k(dst3, zeros_hbm, jnp.ones((_CHUNK,), jnp.float32))


_NBUF = 2
_LOOK = 1
_IDXB = 2 * _NBUF


def _sc_edge_agg(h, ei4, zeros_hbm, npad, nch):
    n, d = h.shape
    rps = npad // _NS
    assert nch % _IDXB == 0

    @functools.partial(
        pl.kernel,
        out_type=jax.ShapeDtypeStruct((_NC, npad, d), jnp.float32),
        mesh=_sc_mesh(),
        scratch_types=[pltpu.VMEM((_CHUNK, d), jnp.float32)] * _NBUF
        + [pltpu.VMEM((2, _CHUNK), jnp.int32)] * _IDXB
        + [pltpu.VMEM_SHARED((npad, d), jnp.float32)]
        + [pltpu.SemaphoreType.DMA] * (2 * _NBUF + _IDXB),
    )
    def k(h_hbm, ei_hbm, zeros_h, out_hbm, *rest):
        rows = rest[:_NBUF]
        idxs = rest[_NBUF : _NBUF + _IDXB]
        acc = rest[_NBUF + _IDXB]
        sems = rest[_NBUF + _IDXB + 1 :]
        gsem = sems[:_NBUF]
        ssem = sems[_NBUF : 2 * _NBUF]
        isem = sems[2 * _NBUF :]
        c = lax.axis_index("c")
        s = lax.axis_index("s")
        wid = s * _NC + c
        r0 = s * rps
        pltpu.sync_copy(zeros_h, acc.at[pl.ds(r0, rps)])

        def idx_start(chunk, q):
            pltpu.async_copy(ei_hbm.at[wid, chunk], idxs[q], isem[q])

        def idx_wait(q):
            pltpu.make_async_copy(ei_hbm.at[0, 0], idxs[q], isem[q]).wait()

        def gather_start(q, b):
            pltpu.async_copy(h_hbm.at[idxs[q].at[0]], rows[b], gsem[b])

        def gather_wait(b):
            pltpu.make_async_copy(h_hbm.at[pl.ds(0, _CHUNK)], rows[b], gsem[b]).wait()

        def scatter_start(q, b):
            pltpu.async_copy(rows[b], acc.at[idxs[q].at[1]], ssem[b], add=True)

        def scatter_wait(b):
            pltpu.make_async_copy(rows[b], acc.at[pl.ds(0, _CHUNK)], ssem[b]).wait()

        for j in range(_NBUF):
            idx_start(j, j)
        plsc.subcore_barrier()

        def emit_iter(i, u, first):
            b = u % _NBUF
            q = u % _IDXB
            idx_wait(q)
            if not first or u >= _NBUF:
                scatter_wait(b)
            gather_start(q, b)
            idx_start(lax.rem(i + _NBUF, nch), (u + _NBUF) % _IDXB)
            if not first or u >= _LOOK:
                bl = (u - _LOOK) % _NBUF
                gather_wait(bl)
                scatter_start((u - _LOOK) % _IDXB, bl)

        for u in range(_IDXB):
            emit_iter(jnp.int32(u), u, True)

        def blk(t, carry):
            for u in range(_IDXB):
                emit_iter(t * _IDXB + u, u, False)
            return carry

        lax.fori_loop(1, nch // _IDXB, blk, 0)
        for j in range(_LOOK):
            u = nch + j
            bl = (u - _LOOK) % _NBUF
            gather_wait(bl)
            scatter_start((u - _LOOK) % _IDXB, bl)
        for b in range(_NBUF):
            scatter_wait(b)
        for j in range(_NBUF):
            idx_wait((nch + j) % _IDXB)
        plsc.subcore_barrier()
        pltpu.sync_copy(acc.at[pl.ds(r0, rps)], out_hbm.at[c, pl.ds(r0, rps)])

    return k(h, ei4, zeros_hbm)


def _tc_layer1(x, w1, d0, d1, bm=1000):
    m, d = x.shape

    def body(x_ref, w_ref, d0_ref, d1_ref, o_ref):
        dis = lax.rsqrt(d0_ref[...] + d1_ref[...] + 1.0)
        o_ref[...] = (
            jnp.dot(x_ref[...], w_ref[...], preferred_element_type=jnp.float32) * dis
        )

    return pl.pallas_call(
        body,
        grid=(m // bm,),
        in_specs=[
            pl.BlockSpec((bm, d), lambda i: (i, 0)),
            pl.BlockSpec((d, d), lambda i: (0, 0)),
            pl.BlockSpec((bm, 1), lambda i: (i, 0)),
            pl.BlockSpec((bm, 1), lambda i: (i, 0)),
        ],
        out_specs=pl.BlockSpec((bm, d), lambda i: (i, 0)),
        out_shape=jax.ShapeDtypeStruct((m, d), jnp.float32),
    )(x, w1, d0, d1)


def _tc_layer2(p0, p1, h1p, d0, d1, b1, w2, bm=1000):
    m, d = h1p.shape

    def body(p0_ref, p1_ref, h_ref, d0_ref, d1_ref, b_ref, w_ref, o_ref):
        dis = lax.rsqrt(d0_ref[...] + d1_ref[...] + 1.0)
        z = dis * (p0_ref[...] + p1_ref[...] + h_ref[...]) + b_ref[...]
        z = jnp.maximum(z, 0.0)
        o_ref[...] = (
            jnp.dot(z, w_ref[...], preferred_element_type=jnp.float32) * dis
        )

    row = pl.BlockSpec((bm, d), lambda i: (i, 0))
    return pl.pallas_call(
        body,
        grid=(m // bm,),
        in_specs=[
            row,
            row,
            row,
            pl.BlockSpec((bm, 1), lambda i: (i, 0)),
            pl.BlockSpec((bm, 1), lambda i: (i, 0)),
            pl.BlockSpec((1, d), lambda i: (0, 0)),
            pl.BlockSpec((d, d), lambda i: (0, 0)),
        ],
        out_specs=row,
        out_shape=jax.ShapeDtypeStruct((m, d), jnp.float32),
    )(p0, p1, h1p, d0, d1, b1, w2)


def _tc_layer3(q0, q1, h2p, d0, d1, b2, bm=1000):
    m, d = h2p.shape

    def body(q0_ref, q1_ref, h_ref, d0_ref, d1_ref, b_ref, o_ref):
        dis = lax.rsqrt(d0_ref[...] + d1_ref[...] + 1.0)
        o_ref[...] = dis * (q0_ref[...] + q1_ref[...] + h_ref[...]) + b_ref[...]

    row = pl.BlockSpec((bm, d), lambda i: (i, 0))
    return pl.pallas_call(
        body,
        grid=(m // bm,),
        in_specs=[
            row,
            row,
            row,
            pl.BlockSpec((bm, 1), lambda i: (i, 0)),
            pl.BlockSpec((bm, 1), lambda i: (i, 0)),
            pl.BlockSpec((1, d), lambda i: (0, 0)),
        ],
        out_specs=row,
        out_shape=jax.ShapeDtypeStruct((m, d), jnp.float32),
    )(q0, q1, h2p, d0, d1, b2)


def kernel(x, edge_index, W1, b1, W2, b2):
    n, d = x.shape
    e = edge_index.shape[1]

    nch = _IDXB * (-(-e // (_NW * _CHUNK * _IDXB)))
    epw = nch * _CHUNK
    ep = epw * _NW
    npad = 256 * (-(-(n + 1) // 256))

    src = edge_index[0].astype(jnp.int32)
    dst = edge_index[1].astype(jnp.int32)
    pad = ep - e
    pad_src = jnp.arange(pad, dtype=jnp.int32) % n
    pad_dst = n + jnp.arange(pad, dtype=jnp.int32) % (npad - n)
    src3 = jnp.concatenate([src, pad_src]).reshape(_NW, nch, _CHUNK)
    dst3 = jnp.concatenate([dst, pad_dst]).reshape(_NW, nch, _CHUNK)
    ei4 = jnp.stack([src3, dst3], axis=2)

    rps = npad // _NS
    zeros_1d = jnp.zeros((npad,), jnp.float32)
    zeros_d = jnp.zeros((rps, d), jnp.float32)

    degp = _sc_degree(dst3, zeros_1d, npad, nch)
    d0 = degp[0, :n].reshape(n, 1)
    d1 = degp[1, :n].reshape(n, 1)

    h1p = _tc_layer1(x, W1, d0, d1)

    p = _sc_edge_agg(h1p, ei4, zeros_d, npad, nch)
    h2p = _tc_layer2(p[0, :n, :], p[1, :n, :], h1p, d0, d1, b1.reshape(1, d), W2)

    q = _sc_edge_agg(h2p, ei4, zeros_d, npad, nch)
    return _tc_layer3(q[0, :n, :], q[1, :n, :], h2p, d0, d1, b2.reshape(1, d))


# speedup vs baseline: 3.0069x; 1.0082x over previous
import functools

import jax
import jax.numpy as jnp
from jax import lax
from jax.experimental import pallas as pl
from jax.experimental.pallas import tpu as pltpu
from jax.experimental.pallas import tpu_sc as plsc

_NC = 2
_NS = 16
_NW = _NC * _NS
_CHUNK = 128


def _sc_mesh():
    return plsc.VectorSubcoreMesh(
        core_axis_name="c", subcore_axis_name="s", num_cores=_NC, num_subcores=_NS
    )


def _sc_degree(dst3, zeros_hbm, npad, nch):
    rps = npad // _NS

    @functools.partial(
        pl.kernel,
        out_type=jax.ShapeDtypeStruct((_NC, npad), jnp.float32),
        mesh=_sc_mesh(),
        scratch_types=[
            pltpu.VMEM((nch, _CHUNK), jnp.int32),
            pltpu.VMEM((_CHUNK,), jnp.float32),
            pltpu.VMEM_SHARED((npad,), jnp.float32),
        ],
    )
    def k(dst_hbm, zeros_h, ones_h, out_hbm, didx, ones_v, acc):
        c = lax.axis_index("c")
        s = lax.axis_index("s")
        wid = s * _NC + c
        r0 = s * rps
        pltpu.sync_copy(zeros_h.at[pl.ds(r0, rps)], acc.at[pl.ds(r0, rps)])
        pltpu.sync_copy(ones_h, ones_v)
        pltpu.sync_copy(dst_hbm.at[wid], didx)
        plsc.subcore_barrier()

        def body(i, carry):
            pltpu.sync_copy(ones_v, acc.at[didx.at[i]], add=True)
            return carry

        lax.fori_loop(0, nch, body, 0)
        plsc.subcore_barrier()
        pltpu.sync_copy(acc.at[pl.ds(r0, rps)], out_hbm.at[c, pl.ds(r0, rps)])

    return k(dst3, zeros_hbm, jnp.ones((_CHUNK,), jnp.float32))


_NBUF = 2
_LOOK = 1
_IDXB = 2 * _NBUF


def _sc_edge_agg(h, ei4, zeros_hbm, npad, nch):
    n, d = h.shape
    rps = npad // _NS
    assert nch % _IDXB == 0

    @functools.partial(
        pl.kernel,
        out_type=jax.ShapeDtypeStruct((_NC, npad, d), jnp.float32),
        mesh=_sc_mesh(),
        scratch_types=[pltpu.VMEM((_CHUNK, d), jnp.float32)] * _NBUF
        + [pltpu.VMEM((2, _CHUNK), jnp.int32)] * _IDXB
        + [pltpu.VMEM_SHARED((npad, d), jnp.float32)]
        + [pltpu.SemaphoreType.DMA] * (2 * _NBUF + _IDXB),
    )
    def k(h_hbm, ei_hbm, zeros_h, out_hbm, *rest):
        rows = rest[:_NBUF]
        idxs = rest[_NBUF : _NBUF + _IDXB]
        acc = rest[_NBUF + _IDXB]
        sems = rest[_NBUF + _IDXB + 1 :]
        gsem = sems[:_NBUF]
        ssem = sems[_NBUF : 2 * _NBUF]
        isem = sems[2 * _NBUF :]
        c = lax.axis_index("c")
        s = lax.axis_index("s")
        wid = s * _NC + c
        r0 = s * rps
        pltpu.sync_copy(zeros_h, acc.at[pl.ds(r0, rps)])

        def idx_start(chunk, q):
            pltpu.async_copy(ei_hbm.at[wid, chunk], idxs[q], isem[q])

        def idx_wait(q):
            pltpu.make_async_copy(ei_hbm.at[0, 0], idxs[q], isem[q]).wait()

        def gather_start(q, b):
            pltpu.async_copy(h_hbm.at[idxs[q].at[0]], rows[b], gsem[b])

        def gather_wait(b):
            pltpu.make_async_copy(h_hbm.at[pl.ds(0, _CHUNK)], rows[b], gsem[b]).wait()

        def scatter_start(q, b):
            pltpu.async_copy(rows[b], acc.at[idxs[q].at[1]], ssem[b], add=True)

        def scatter_wait(b):
            pltpu.make_async_copy(rows[b], acc.at[pl.ds(0, _CHUNK)], ssem[b]).wait()

        for j in range(_NBUF):
            idx_start(j, j)
        plsc.subcore_barrier()

        def emit_iter(i, u, first):
            b = u % _NBUF
            q = u % _IDXB
            idx_wait(q)
            if not first or u >= _NBUF:
                scatter_wait(b)
            gather_start(q, b)
            idx_start(lax.rem(i + _NBUF, nch), (u + _NBUF) % _IDXB)
            if not first or u >= _LOOK:
                bl = (u - _LOOK) % _NBUF
                gather_wait(bl)
                scatter_start((u - _LOOK) % _IDXB, bl)

        for u in range(_IDXB):
            emit_iter(jnp.int32(u), u, True)

        def blk(t, carry):
            for u in range(_IDXB):
                emit_iter(t * _IDXB + u, u, False)
            return carry

        lax.fori_loop(1, nch // _IDXB, blk, 0)
        for j in range(_LOOK):
            u = nch + j
            bl = (u - _LOOK) % _NBUF
            gather_wait(bl)
            scatter_start((u - _LOOK) % _IDXB, bl)
        for b in range(_NBUF):
            scatter_wait(b)
        for j in range(_NBUF):
            idx_wait((nch + j) % _IDXB)
        plsc.subcore_barrier()
        pltpu.sync_copy(acc.at[pl.ds(r0, rps)], out_hbm.at[c, pl.ds(r0, rps)])

    return k(h, ei4, zeros_hbm)


def _tc_matmul(x, w1, bm=1000):
    m, d = x.shape

    def body(x_ref, w_ref, o_ref):
        o_ref[...] = jnp.dot(
            x_ref[...], w_ref[...], preferred_element_type=jnp.float32
        )

    return pl.pallas_call(
        body,
        grid=(m // bm,),
        in_specs=[
            pl.BlockSpec((bm, d), lambda i: (i, 0)),
            pl.BlockSpec((d, d), lambda i: (0, 0)),
        ],
        out_specs=pl.BlockSpec((bm, d), lambda i: (i, 0)),
        out_shape=jax.ShapeDtypeStruct((m, d), jnp.float32),
    )(x, w1)


def _tc_scale(u, d0, d1, bm=1000):
    m, d = u.shape

    def body(u_ref, d0_ref, d1_ref, o_ref):
        dis = lax.rsqrt(d0_ref[...] + d1_ref[...] + 1.0)
        o_ref[...] = u_ref[...] * dis

    return pl.pallas_call(
        body,
        grid=(m // bm,),
        in_specs=[
            pl.BlockSpec((bm, d), lambda i: (i, 0)),
            pl.BlockSpec((bm, 1), lambda i: (i, 0)),
            pl.BlockSpec((bm, 1), lambda i: (i, 0)),
        ],
        out_specs=pl.BlockSpec((bm, d), lambda i: (i, 0)),
        out_shape=jax.ShapeDtypeStruct((m, d), jnp.float32),
    )(u, d0, d1)


def _tc_layer2(p0, p1, h1p, d0, d1, b1, w2, bm=1000):
    m, d = h1p.shape

    def body(p0_ref, p1_ref, h_ref, d0_ref, d1_ref, b_ref, w_ref, o_ref):
        dis = lax.rsqrt(d0_ref[...] + d1_ref[...] + 1.0)
        z = dis * (p0_ref[...] + p1_ref[...] + h_ref[...]) + b_ref[...]
        z = jnp.maximum(z, 0.0)
        o_ref[...] = (
            jnp.dot(z, w_ref[...], preferred_element_type=jnp.float32) * dis
        )

    row = pl.BlockSpec((bm, d), lambda i: (i, 0))
    return pl.pallas_call(
        body,
        grid=(m // bm,),
        in_specs=[
            row,
            row,
            row,
            pl.BlockSpec((bm, 1), lambda i: (i, 0)),
            pl.BlockSpec((bm, 1), lambda i: (i, 0)),
            pl.BlockSpec((1, d), lambda i: (0, 0)),
            pl.BlockSpec((d, d), lambda i: (0, 0)),
        ],
        out_specs=row,
        out_shape=jax.ShapeDtypeStruct((m, d), jnp.float32),
    )(p0, p1, h1p, d0, d1, b1, w2)


def _tc_layer3(q0, q1, h2p, d0, d1, b2, bm=1000):
    m, d = h2p.shape

    def body(q0_ref, q1_ref, h_ref, d0_ref, d1_ref, b_ref, o_ref):
        dis = lax.rsqrt(d0_ref[...] + d1_ref[...] + 1.0)
        o_ref[...] = dis * (q0_ref[...] + q1_ref[...] + h_ref[...]) + b_ref[...]

    row = pl.BlockSpec((bm, d), lambda i: (i, 0))
    return pl.pallas_call(
        body,
        grid=(m // bm,),
        in_specs=[
            row,
            row,
            row,
            pl.BlockSpec((bm, 1), lambda i: (i, 0)),
            pl.BlockSpec((bm, 1), lambda i: (i, 0)),
            pl.BlockSpec((1, d), lambda i: (0, 0)),
        ],
        out_specs=row,
        out_shape=jax.ShapeDtypeStruct((m, d), jnp.float32),
    )(q0, q1, h2p, d0, d1, b2)


def kernel(x, edge_index, W1, b1, W2, b2):
    n, d = x.shape
    e = edge_index.shape[1]

    nch = _IDXB * (-(-e // (_NW * _CHUNK * _IDXB)))
    epw = nch * _CHUNK
    ep = epw * _NW
    npad = 256 * (-(-(n + 1) // 256))

    src = edge_index[0].astype(jnp.int32)
    dst = edge_index[1].astype(jnp.int32)
    pad = ep - e
    pad_src = jnp.arange(pad, dtype=jnp.int32) % n
    pad_dst = n + jnp.arange(pad, dtype=jnp.int32) % (npad - n)
    src3 = jnp.concatenate([src, pad_src]).reshape(_NW, nch, _CHUNK)
    dst3 = jnp.concatenate([dst, pad_dst]).reshape(_NW, nch, _CHUNK)
    ei4 = jnp.stack([src3, dst3], axis=2)

    rps = npad // _NS
    zeros_1d = jnp.zeros((npad,), jnp.float32)
    zeros_d = jnp.zeros((rps, d), jnp.float32)

    u1 = _tc_matmul(x, W1)
    degp = _sc_degree(dst3, zeros_1d, npad, nch)
    d0 = degp[0, :n].reshape(n, 1)
    d1 = degp[1, :n].reshape(n, 1)

    h1p = _tc_scale(u1, d0, d1)

    p = _sc_edge_agg(h1p, ei4, zeros_d, npad, nch)
    h2p = _tc_layer2(p[0, :n, :], p[1, :n, :], h1p, d0, d1, b1.reshape(1, d), W2)

    q = _sc_edge_agg(h2p, ei4, zeros_d, npad, nch)
    return _tc_layer3(q[0, :n, :], q[1, :n, :], h2p, d0, d1, b2.reshape(1, d))


# speedup vs baseline: 3.0175x; 1.0035x over previous
import functools

import jax
import jax.numpy as jnp
from jax import lax
from jax.experimental import pallas as pl
from jax.experimental.pallas import tpu as pltpu
from jax.experimental.pallas import tpu_sc as plsc

_NC = 2
_NS = 16
_NW = _NC * _NS
_CHUNK = 128


def _sc_mesh():
    return plsc.VectorSubcoreMesh(
        core_axis_name="c", subcore_axis_name="s", num_cores=_NC, num_subcores=_NS
    )


def _sc_degree(dst3, zeros_hbm, npad, nch):
    rps = npad // _NS

    @functools.partial(
        pl.kernel,
        out_type=jax.ShapeDtypeStruct((_NC, npad), jnp.float32),
        mesh=_sc_mesh(),
        scratch_types=[
            pltpu.VMEM((nch, _CHUNK), jnp.int32),
            pltpu.VMEM((_CHUNK,), jnp.float32),
            pltpu.VMEM_SHARED((npad,), jnp.float32),
            pltpu.SemaphoreType.DMA,
        ],
    )
    def k(dst_hbm, zeros_h, ones_h, out_hbm, didx, ones_v, acc, sem):
        c = lax.axis_index("c")
        s = lax.axis_index("s")
        wid = s * _NC + c
        r0 = s * rps
        pltpu.sync_copy(zeros_h.at[pl.ds(r0, rps)], acc.at[pl.ds(r0, rps)])
        pltpu.sync_copy(ones_h, ones_v)
        pltpu.sync_copy(dst_hbm.at[wid], didx)
        plsc.subcore_barrier()

        def body(i, carry):
            pltpu.async_copy(ones_v, acc.at[didx.at[i]], sem, add=True)
            return carry

        lax.fori_loop(0, nch, body, 0)

        def drain(i, carry):
            pltpu.make_async_copy(ones_v, acc.at[pl.ds(0, _CHUNK)], sem).wait()
            return carry

        lax.fori_loop(0, nch, drain, 0)
        plsc.subcore_barrier()
        pltpu.sync_copy(acc.at[pl.ds(r0, rps)], out_hbm.at[c, pl.ds(r0, rps)])

    return k(dst3, zeros_hbm, jnp.ones((_CHUNK,), jnp.float32))


_NBUF = 2
_LOOK = 1
_IDXB = 2 * _NBUF


def _sc_edge_agg(h, ei4, zeros_hbm, npad, nch):
    n, d = h.shape
    rps = npad // _NS
    assert nch % _IDXB == 0

    @functools.partial(
        pl.kernel,
        out_type=jax.ShapeDtypeStruct((_NC, npad, d), jnp.float32),
        mesh=_sc_mesh(),
        scratch_types=[pltpu.VMEM((_CHUNK, d), jnp.float32)] * _NBUF
        + [pltpu.VMEM((2, _CHUNK), jnp.int32)] * _IDXB
        + [pltpu.VMEM_SHARED((npad, d), jnp.float32)]
        + [pltpu.SemaphoreType.DMA] * (2 * _NBUF + _IDXB),
    )
    def k(h_hbm, ei_hbm, zeros_h, out_hbm, *rest):
        rows = rest[:_NBUF]
        idxs = rest[_NBUF : _NBUF + _IDXB]
        acc = rest[_NBUF + _IDXB]
        sems = rest[_NBUF + _IDXB + 1 :]
        gsem = sems[:_NBUF]
        ssem = sems[_NBUF : 2 * _NBUF]
        isem = sems[2 * _NBUF :]
        c = lax.axis_index("c")
        s = lax.axis_index("s")
        wid = s * _NC + c
        r0 = s * rps
        pltpu.sync_copy(zeros_h, acc.at[pl.ds(r0, rps)])

        def idx_start(chunk, q):
            pltpu.async_copy(ei_hbm.at[wid, chunk], idxs[q], isem[q])

        def idx_wait(q):
            pltpu.make_async_copy(ei_hbm.at[0, 0], idxs[q], isem[q]).wait()

        def gather_start(q, b):
            pltpu.async_copy(h_hbm.at[idxs[q].at[0]], rows[b], gsem[b])

        def gather_wait(b):
            pltpu.make_async_copy(h_hbm.at[pl.ds(0, _CHUNK)], rows[b], gsem[b]).wait()

        def scatter_start(q, b):
            pltpu.async_copy(rows[b], acc.at[idxs[q].at[1]], ssem[b], add=True)

        def scatter_wait(b):
            pltpu.make_async_copy(rows[b], acc.at[pl.ds(0, _CHUNK)], ssem[b]).wait()

        for j in range(_NBUF):
            idx_start(j, j)
        plsc.subcore_barrier()

        def emit_iter(i, u, first):
            b = u % _NBUF
            q = u % _IDXB
            idx_wait(q)
            if not first or u >= _NBUF:
                scatter_wait(b)
            gather_start(q, b)
            idx_start(lax.rem(i + _NBUF, nch), (u + _NBUF) % _IDXB)
            if not first or u >= _LOOK:
                bl = (u - _LOOK) % _NBUF
                gather_wait(bl)
                scatter_start((u - _LOOK) % _IDXB, bl)

        for u in range(_IDXB):
            emit_iter(jnp.int32(u), u, True)

        def blk(t, carry):
            for u in range(_IDXB):
                emit_iter(t * _IDXB + u, u, False)
            return carry

        lax.fori_loop(1, nch // _IDXB, blk, 0)
        for j in range(_LOOK):
            u = nch + j
            bl = (u - _LOOK) % _NBUF
            gather_wait(bl)
            scatter_start((u - _LOOK) % _IDXB, bl)
        for b in range(_NBUF):
            scatter_wait(b)
        for j in range(_NBUF):
            idx_wait((nch + j) % _IDXB)
        plsc.subcore_barrier()
        pltpu.sync_copy(acc.at[pl.ds(r0, rps)], out_hbm.at[c, pl.ds(r0, rps)])

    return k(h, ei4, zeros_hbm)


def _tc_matmul(x, w1, bm=1000):
    m, d = x.shape

    def body(x_ref, w_ref, o_ref):
        o_ref[...] = jnp.dot(
            x_ref[...], w_ref[...], preferred_element_type=jnp.float32
        )

    return pl.pallas_call(
        body,
        grid=(m // bm,),
        in_specs=[
            pl.BlockSpec((bm, d), lambda i: (i, 0)),
            pl.BlockSpec((d, d), lambda i: (0, 0)),
        ],
        out_specs=pl.BlockSpec((bm, d), lambda i: (i, 0)),
        out_shape=jax.ShapeDtypeStruct((m, d), jnp.float32),
    )(x, w1)


def _tc_scale(u, d0, d1, bm=1000):
    m, d = u.shape

    def body(u_ref, d0_ref, d1_ref, o_ref):
        dis = lax.rsqrt(d0_ref[...] + d1_ref[...] + 1.0)
        o_ref[...] = u_ref[...] * dis

    return pl.pallas_call(
        body,
        grid=(m // bm,),
        in_specs=[
            pl.BlockSpec((bm, d), lambda i: (i, 0)),
            pl.BlockSpec((bm, 1), lambda i: (i, 0)),
            pl.BlockSpec((bm, 1), lambda i: (i, 0)),
        ],
        out_specs=pl.BlockSpec((bm, d), lambda i: (i, 0)),
        out_shape=jax.ShapeDtypeStruct((m, d), jnp.float32),
    )(u, d0, d1)


def _tc_layer2(p0, p1, h1p, d0, d1, b1, w2, bm=1000):
    m, d = h1p.shape

    def body(p0_ref, p1_ref, h_ref, d0_ref, d1_ref, b_ref, w_ref, o_ref):
        dis = lax.rsqrt(d0_ref[...] + d1_ref[...] + 1.0)
        z = dis * (p0_ref[...] + p1_ref[...] + h_ref[...]) + b_ref[...]
        z = jnp.maximum(z, 0.0)
        o_ref[...] = (
            jnp.dot(z, w_ref[...], preferred_element_type=jnp.float32) * dis
        )

    row = pl.BlockSpec((bm, d), lambda i: (i, 0))
    return pl.pallas_call(
        body,
        grid=(m // bm,),
        in_specs=[
            row,
            row,
            row,
            pl.BlockSpec((bm, 1), lambda i: (i, 0)),
            pl.BlockSpec((bm, 1), lambda i: (i, 0)),
            pl.BlockSpec((1, d), lambda i: (0, 0)),
            pl.BlockSpec((d, d), lambda i: (0, 0)),
        ],
        out_specs=row,
        out_shape=jax.ShapeDtypeStruct((m, d), jnp.float32),
    )(p0, p1, h1p, d0, d1, b1, w2)


def _tc_layer3(q0, q1, h2p, d0, d1, b2, bm=1000):
    m, d = h2p.shape

    def body(q0_ref, q1_ref, h_ref, d0_ref, d1_ref, b_ref, o_ref):
        dis = lax.rsqrt(d0_ref[...] + d1_ref[...] + 1.0)
        o_ref[...] = dis * (q0_ref[...] + q1_ref[...] + h_ref[...]) + b_ref[...]

    row = pl.BlockSpec((bm, d), lambda i: (i, 0))
    return pl.pallas_call(
        body,
        grid=(m // bm,),
        in_specs=[
            row,
            row,
            row,
            pl.BlockSpec((bm, 1), lambda i: (i, 0)),
            pl.BlockSpec((bm, 1), lambda i: (i, 0)),
            pl.BlockSpec((1, d), lambda i: (0, 0)),
        ],
        out_specs=row,
        out_shape=jax.ShapeDtypeStruct((m, d), jnp.float32),
    )(q0, q1, h2p, d0, d1, b2)


def kernel(x, edge_index, W1, b1, W2, b2):
    n, d = x.shape
    e = edge_index.shape[1]

    nch = _IDXB * (-(-e // (_NW * _CHUNK * _IDXB)))
    epw = nch * _CHUNK
    ep = epw * _NW
    npad = 256 * (-(-(n + 1) // 256))

    src = edge_index[0].astype(jnp.int32)
    dst = edge_index[1].astype(jnp.int32)
    pad = ep - e
    pad_src = jnp.arange(pad, dtype=jnp.int32) % n
    pad_dst = n + jnp.arange(pad, dtype=jnp.int32) % (npad - n)
    src3 = jnp.concatenate([src, pad_src]).reshape(_NW, nch, _CHUNK)
    dst3 = jnp.concatenate([dst, pad_dst]).reshape(_NW, nch, _CHUNK)
    ei4 = jnp.stack([src3, dst3], axis=2)

    rps = npad // _NS
    zeros_1d = jnp.zeros((npad,), jnp.float32)
    zeros_d = jnp.zeros((rps, d), jnp.float32)

    u1 = _tc_matmul(x, W1)
    degp = _sc_degree(dst3, zeros_1d, npad, nch)
    d0 = degp[0, :n].reshape(n, 1)
    d1 = degp[1, :n].reshape(n, 1)

    h1p = _tc_scale(u1, d0, d1)

    p = _sc_edge_agg(h1p, ei4, zeros_d, npad, nch)
    h2p = _tc_layer2(p[0, :n, :], p[1, :n, :], h1p, d0, d1, b1.reshape(1, d), W2)

    q = _sc_edge_agg(h2p, ei4, zeros_d, npad, nch)
    return _tc_layer3(q[0, :n, :], q[1, :n, :], h2p, d0, d1, b2.reshape(1, d))


# speedup vs baseline: 3.0403x; 1.0076x over previous
import functools

import jax
import jax.numpy as jnp
from jax import lax
from jax.experimental import pallas as pl
from jax.experimental.pallas import tpu as pltpu
from jax.experimental.pallas import tpu_sc as plsc

_NC = 2
_NS = 16
_NW = _NC * _NS
_CHUNK = 120


def _sc_mesh():
    return plsc.VectorSubcoreMesh(
        core_axis_name="c", subcore_axis_name="s", num_cores=_NC, num_subcores=_NS
    )


def _sc_degree(dst3, zeros_hbm, npad, nch):
    rps = npad // _NS

    @functools.partial(
        pl.kernel,
        out_type=jax.ShapeDtypeStruct((_NC, npad), jnp.float32),
        mesh=_sc_mesh(),
        scratch_types=[
            pltpu.VMEM((nch, _CHUNK), jnp.int32),
            pltpu.VMEM((_CHUNK,), jnp.float32),
            pltpu.VMEM_SHARED((npad,), jnp.float32),
            pltpu.SemaphoreType.DMA,
        ],
    )
    def k(dst_hbm, zeros_h, ones_h, out_hbm, didx, ones_v, acc, sem):
        c = lax.axis_index("c")
        s = lax.axis_index("s")
        wid = s * _NC + c
        r0 = s * rps
        pltpu.sync_copy(zeros_h.at[pl.ds(r0, rps)], acc.at[pl.ds(r0, rps)])
        pltpu.sync_copy(ones_h, ones_v)
        pltpu.sync_copy(dst_hbm.at[wid], didx)
        plsc.subcore_barrier()

        def body(i, carry):
            pltpu.async_copy(ones_v, acc.at[didx.at[i]], sem, add=True)
            return carry

        lax.fori_loop(0, nch, body, 0)

        def drain(i, carry):
            pltpu.make_async_copy(ones_v, acc.at[pl.ds(0, _CHUNK)], sem).wait()
            return carry

        lax.fori_loop(0, nch, drain, 0)
        plsc.subcore_barrier()
        pltpu.sync_copy(acc.at[pl.ds(r0, rps)], out_hbm.at[c, pl.ds(r0, rps)])

    return k(dst3, zeros_hbm, jnp.ones((_CHUNK,), jnp.float32))


_NBUF = 3
_LOOK = 1
_IDXB = 2 * _NBUF


def _sc_edge_agg(h, ei4, zeros_hbm, npad, nch):
    n, d = h.shape
    rps = npad // _NS
    assert nch % _IDXB == 0

    @functools.partial(
        pl.kernel,
        out_type=jax.ShapeDtypeStruct((_NC, npad, d), jnp.float32),
        mesh=_sc_mesh(),
        scratch_types=[pltpu.VMEM((_CHUNK, d), jnp.float32)] * _NBUF
        + [pltpu.VMEM((2, _CHUNK), jnp.int32)] * _IDXB
        + [pltpu.VMEM_SHARED((npad, d), jnp.float32)]
        + [pltpu.SemaphoreType.DMA] * (2 * _NBUF + _IDXB),
    )
    def k(h_hbm, ei_hbm, zeros_h, out_hbm, *rest):
        rows = rest[:_NBUF]
        idxs = rest[_NBUF : _NBUF + _IDXB]
        acc = rest[_NBUF + _IDXB]
        sems = rest[_NBUF + _IDXB + 1 :]
        gsem = sems[:_NBUF]
        ssem = sems[_NBUF : 2 * _NBUF]
        isem = sems[2 * _NBUF :]
        c = lax.axis_index("c")
        s = lax.axis_index("s")
        wid = s * _NC + c
        r0 = s * rps
        pltpu.sync_copy(zeros_h, acc.at[pl.ds(r0, rps)])

        def idx_start(chunk, q):
            pltpu.async_copy(ei_hbm.at[wid, chunk], idxs[q], isem[q])

        def idx_wait(q):
            pltpu.make_async_copy(ei_hbm.at[0, 0], idxs[q], isem[q]).wait()

        def gather_start(q, b):
            pltpu.async_copy(h_hbm.at[idxs[q].at[0]], rows[b], gsem[b])

        def gather_wait(b):
            pltpu.make_async_copy(h_hbm.at[pl.ds(0, _CHUNK)], rows[b], gsem[b]).wait()

        def scatter_start(q, b):
            pltpu.async_copy(rows[b], acc.at[idxs[q].at[1]], ssem[b], add=True)

        def scatter_wait(b):
            pltpu.make_async_copy(rows[b], acc.at[pl.ds(0, _CHUNK)], ssem[b]).wait()

        for j in range(_NBUF):
            idx_start(j, j)
        plsc.subcore_barrier()

        def emit_iter(i, u, first):
            b = u % _NBUF
            q = u % _IDXB
            idx_wait(q)
            if not first or u >= _NBUF:
                scatter_wait(b)
            gather_start(q, b)
            idx_start(lax.rem(i + _NBUF, nch), (u + _NBUF) % _IDXB)
            if not first or u >= _LOOK:
                bl = (u - _LOOK) % _NBUF
                gather_wait(bl)
                scatter_start((u - _LOOK) % _IDXB, bl)

        for u in range(_IDXB):
            emit_iter(jnp.int32(u), u, True)

        def blk(t, carry):
            for u in range(_IDXB):
                emit_iter(t * _IDXB + u, u, False)
            return carry

        lax.fori_loop(1, nch // _IDXB, blk, 0)
        for j in range(_LOOK):
            u = nch + j
            bl = (u - _LOOK) % _NBUF
            gather_wait(bl)
            scatter_start((u - _LOOK) % _IDXB, bl)
        for b in range(_NBUF):
            scatter_wait(b)
        for j in range(_NBUF):
            idx_wait((nch + j) % _IDXB)
        plsc.subcore_barrier()
        pltpu.sync_copy(acc.at[pl.ds(r0, rps)], out_hbm.at[c, pl.ds(r0, rps)])

    return k(h, ei4, zeros_hbm)


def _tc_matmul(x, w1, bm=1000):
    m, d = x.shape

    def body(x_ref, w_ref, o_ref):
        o_ref[...] = jnp.dot(
            x_ref[...], w_ref[...], preferred_element_type=jnp.float32
        )

    return pl.pallas_call(
        body,
        grid=(m // bm,),
        in_specs=[
            pl.BlockSpec((bm, d), lambda i: (i, 0)),
            pl.BlockSpec((d, d), lambda i: (0, 0)),
        ],
        out_specs=pl.BlockSpec((bm, d), lambda i: (i, 0)),
        out_shape=jax.ShapeDtypeStruct((m, d), jnp.float32),
    )(x, w1)


def _tc_scale(u, d0, d1, bm=1000):
    m, d = u.shape

    def body(u_ref, d0_ref, d1_ref, o_ref):
        dis = lax.rsqrt(d0_ref[...] + d1_ref[...] + 1.0)
        o_ref[...] = u_ref[...] * dis

    return pl.pallas_call(
        body,
        grid=(m // bm,),
        in_specs=[
            pl.BlockSpec((bm, d), lambda i: (i, 0)),
            pl.BlockSpec((bm, 1), lambda i: (i, 0)),
            pl.BlockSpec((bm, 1), lambda i: (i, 0)),
        ],
        out_specs=pl.BlockSpec((bm, d), lambda i: (i, 0)),
        out_shape=jax.ShapeDtypeStruct((m, d), jnp.float32),
    )(u, d0, d1)


def _tc_layer2(p0, p1, h1p, d0, d1, b1, w2, bm=1000):
    m, d = h1p.shape

    def body(p0_ref, p1_ref, h_ref, d0_ref, d1_ref, b_ref, w_ref, o_ref):
        dis = lax.rsqrt(d0_ref[...] + d1_ref[...] + 1.0)
        z = dis * (p0_ref[...] + p1_ref[...] + h_ref[...]) + b_ref[...]
        z = jnp.maximum(z, 0.0)
        o_ref[...] = (
            jnp.dot(z, w_ref[...], preferred_element_type=jnp.float32) * dis
        )

    row = pl.BlockSpec((bm, d), lambda i: (i, 0))
    return pl.pallas_call(
        body,
        grid=(m // bm,),
        in_specs=[
            row,
            row,
            row,
            pl.BlockSpec((bm, 1), lambda i: (i, 0)),
            pl.BlockSpec((bm, 1), lambda i: (i, 0)),
            pl.BlockSpec((1, d), lambda i: (0, 0)),
            pl.BlockSpec((d, d), lambda i: (0, 0)),
        ],
        out_specs=row,
        out_shape=jax.ShapeDtypeStruct((m, d), jnp.float32),
    )(p0, p1, h1p, d0, d1, b1, w2)


def _tc_layer3(q0, q1, h2p, d0, d1, b2, bm=1000):
    m, d = h2p.shape

    def body(q0_ref, q1_ref, h_ref, d0_ref, d1_ref, b_ref, o_ref):
        dis = lax.rsqrt(d0_ref[...] + d1_ref[...] + 1.0)
        o_ref[...] = dis * (q0_ref[...] + q1_ref[...] + h_ref[...]) + b_ref[...]

    row = pl.BlockSpec((bm, d), lambda i: (i, 0))
    return pl.pallas_call(
        body,
        grid=(m // bm,),
        in_specs=[
            row,
            row,
            row,
            pl.BlockSpec((bm, 1), lambda i: (i, 0)),
            pl.BlockSpec((bm, 1), lambda i: (i, 0)),
            pl.BlockSpec((1, d), lambda i: (0, 0)),
        ],
        out_specs=row,
        out_shape=jax.ShapeDtypeStruct((m, d), jnp.float32),
    )(q0, q1, h2p, d0, d1, b2)


def kernel(x, edge_index, W1, b1, W2, b2):
    n, d = x.shape
    e = edge_index.shape[1]

    nch = _IDXB * (-(-e // (_NW * _CHUNK * _IDXB)))
    epw = nch * _CHUNK
    ep = epw * _NW
    npad = 256 * (-(-(n + 1) // 256))

    src = edge_index[0].astype(jnp.int32)
    dst = edge_index[1].astype(jnp.int32)
    pad = ep - e
    pad_src = jnp.arange(pad, dtype=jnp.int32) % n
    pad_dst = n + jnp.arange(pad, dtype=jnp.int32) % (npad - n)
    src3 = jnp.concatenate([src, pad_src]).reshape(_NW, nch, _CHUNK)
    dst3 = jnp.concatenate([dst, pad_dst]).reshape(_NW, nch, _CHUNK)
    ei4 = jnp.stack([src3, dst3], axis=2)

    rps = npad // _NS
    zeros_1d = jnp.zeros((npad,), jnp.float32)
    zeros_d = jnp.zeros((rps, d), jnp.float32)

    u1 = _tc_matmul(x, W1)
    degp = _sc_degree(dst3, zeros_1d, npad, nch)
    d0 = degp[0, :n].reshape(n, 1)
    d1 = degp[1, :n].reshape(n, 1)

    h1p = _tc_scale(u1, d0, d1)

    p = _sc_edge_agg(h1p, ei4, zeros_d, npad, nch)
    h2p = _tc_layer2(p[0, :n, :], p[1, :n, :], h1p, d0, d1, b1.reshape(1, d), W2)

    q = _sc_edge_agg(h2p, ei4, zeros_d, npad, nch)
    return _tc_layer3(q[0, :n, :], q[1, :n, :], h2p, d0, d1, b2.reshape(1, d))


# speedup vs baseline: 3.1855x; 1.0478x over previous
import functools

import jax
import jax.numpy as jnp
from jax import lax
from jax.experimental import pallas as pl
from jax.experimental.pallas import tpu as pltpu
from jax.experimental.pallas import tpu_sc as plsc

_NC = 2
_NS = 16
_NW = _NC * _NS
_CHUNK = 120


def _sc_mesh():
    return plsc.VectorSubcoreMesh(
        core_axis_name="c", subcore_axis_name="s", num_cores=_NC, num_subcores=_NS
    )


def _sc_degree(dst3, zeros_hbm, npad, nch):
    rps = npad // _NS

    @functools.partial(
        pl.kernel,
        out_type=jax.ShapeDtypeStruct((_NC, npad), jnp.float32),
        mesh=_sc_mesh(),
        scratch_types=[
            pltpu.VMEM((nch, _CHUNK), jnp.int32),
            pltpu.VMEM((_CHUNK,), jnp.float32),
            pltpu.VMEM_SHARED((npad,), jnp.float32),
            pltpu.SemaphoreType.DMA,
        ],
    )
    def k(dst_hbm, zeros_h, ones_h, out_hbm, didx, ones_v, acc, sem):
        c = lax.axis_index("c")
        s = lax.axis_index("s")
        wid = s * _NC + c
        r0 = s * rps
        pltpu.sync_copy(zeros_h.at[pl.ds(r0, rps)], acc.at[pl.ds(r0, rps)])
        pltpu.sync_copy(ones_h, ones_v)
        pltpu.sync_copy(dst_hbm.at[wid], didx)
        plsc.subcore_barrier()

        def body(i, carry):
            pltpu.async_copy(ones_v, acc.at[didx.at[i]], sem, add=True)
            return carry

        lax.fori_loop(0, nch, body, 0)

        def drain(i, carry):
            pltpu.make_async_copy(ones_v, acc.at[pl.ds(0, _CHUNK)], sem).wait()
            return carry

        lax.fori_loop(0, nch, drain, 0)
        plsc.subcore_barrier()
        pltpu.sync_copy(acc.at[pl.ds(r0, rps)], out_hbm.at[c, pl.ds(r0, rps)])

    return k(dst3, zeros_hbm, jnp.ones((_CHUNK,), jnp.float32))


_NBUF = 3
_LOOK = 2
_IDXB = 2 * _NBUF


def _sc_edge_agg(h, ei4, zeros_hbm, npad, nch):
    n, d = h.shape
    rps = npad // _NS
    assert nch % _IDXB == 0

    @functools.partial(
        pl.kernel,
        out_type=jax.ShapeDtypeStruct((_NC, npad, d), jnp.float32),
        mesh=_sc_mesh(),
        scratch_types=[pltpu.VMEM((_CHUNK, d), jnp.float32)] * _NBUF
        + [pltpu.VMEM((2, _CHUNK), jnp.int32)] * _IDXB
        + [pltpu.VMEM_SHARED((npad, d), jnp.float32)]
        + [pltpu.SemaphoreType.DMA] * (2 * _NBUF + _IDXB),
    )
    def k(h_hbm, ei_hbm, zeros_h, out_hbm, *rest):
        rows = rest[:_NBUF]
        idxs = rest[_NBUF : _NBUF + _IDXB]
        acc = rest[_NBUF + _IDXB]
        sems = rest[_NBUF + _IDXB + 1 :]
        gsem = sems[:_NBUF]
        ssem = sems[_NBUF : 2 * _NBUF]
        isem = sems[2 * _NBUF :]
        c = lax.axis_index("c")
        s = lax.axis_index("s")
        wid = s * _NC + c
        r0 = s * rps
        pltpu.sync_copy(zeros_h, acc.at[pl.ds(r0, rps)])

        def idx_start(chunk, q):
            pltpu.async_copy(ei_hbm.at[wid, chunk], idxs[q], isem[q])

        def idx_wait(q):
            pltpu.make_async_copy(ei_hbm.at[0, 0], idxs[q], isem[q]).wait()

        def gather_start(q, b):
            pltpu.async_copy(h_hbm.at[idxs[q].at[0]], rows[b], gsem[b])

        def gather_wait(b):
            pltpu.make_async_copy(h_hbm.at[pl.ds(0, _CHUNK)], rows[b], gsem[b]).wait()

        def scatter_start(q, b):
            pltpu.async_copy(rows[b], acc.at[idxs[q].at[1]], ssem[b], add=True)

        def scatter_wait(b):
            pltpu.make_async_copy(rows[b], acc.at[pl.ds(0, _CHUNK)], ssem[b]).wait()

        for j in range(_NBUF):
            idx_start(j, j)
        plsc.subcore_barrier()

        def emit_iter(i, u, first):
            b = u % _NBUF
            q = u % _IDXB
            idx_wait(q)
            if not first or u >= _NBUF:
                scatter_wait(b)
            gather_start(q, b)
            idx_start(lax.rem(i + _NBUF, nch), (u + _NBUF) % _IDXB)
            if not first or u >= _LOOK:
                bl = (u - _LOOK) % _NBUF
                gather_wait(bl)
                scatter_start((u - _LOOK) % _IDXB, bl)

        for u in range(_IDXB):
            emit_iter(jnp.int32(u), u, True)

        def blk(t, carry):
            for u in range(_IDXB):
                emit_iter(t * _IDXB + u, u, False)
            return carry

        lax.fori_loop(1, nch // _IDXB, blk, 0)
        for j in range(_LOOK):
            u = nch + j
            bl = (u - _LOOK) % _NBUF
            gather_wait(bl)
            scatter_start((u - _LOOK) % _IDXB, bl)
        for b in range(_NBUF):
            scatter_wait(b)
        for j in range(_NBUF):
            idx_wait((nch + j) % _IDXB)
        plsc.subcore_barrier()
        pltpu.sync_copy(acc.at[pl.ds(r0, rps)], out_hbm.at[c, pl.ds(r0, rps)])

    return k(h, ei4, zeros_hbm)


def _tc_matmul(x, w1, bm=1000):
    m, d = x.shape

    def body(x_ref, w_ref, o_ref):
        o_ref[...] = jnp.dot(
            x_ref[...], w_ref[...], preferred_element_type=jnp.float32
        )

    return pl.pallas_call(
        body,
        grid=(m // bm,),
        in_specs=[
            pl.BlockSpec((bm, d), lambda i: (i, 0)),
            pl.BlockSpec((d, d), lambda i: (0, 0)),
        ],
        out_specs=pl.BlockSpec((bm, d), lambda i: (i, 0)),
        out_shape=jax.ShapeDtypeStruct((m, d), jnp.float32),
    )(x, w1)


def _tc_scale(u, d0, d1, bm=1000):
    m, d = u.shape

    def body(u_ref, d0_ref, d1_ref, o_ref):
        dis = lax.rsqrt(d0_ref[...] + d1_ref[...] + 1.0)
        o_ref[...] = u_ref[...] * dis

    return pl.pallas_call(
        body,
        grid=(m // bm,),
        in_specs=[
            pl.BlockSpec((bm, d), lambda i: (i, 0)),
            pl.BlockSpec((bm, 1), lambda i: (i, 0)),
            pl.BlockSpec((bm, 1), lambda i: (i, 0)),
        ],
        out_specs=pl.BlockSpec((bm, d), lambda i: (i, 0)),
        out_shape=jax.ShapeDtypeStruct((m, d), jnp.float32),
    )(u, d0, d1)


def _tc_layer2(p0, p1, h1p, d0, d1, b1, w2, bm=1000):
    m, d = h1p.shape

    def body(p0_ref, p1_ref, h_ref, d0_ref, d1_ref, b_ref, w_ref, o_ref):
        dis = lax.rsqrt(d0_ref[...] + d1_ref[...] + 1.0)
        z = dis * (p0_ref[...] + p1_ref[...] + h_ref[...]) + b_ref[...]
        z = jnp.maximum(z, 0.0)
        o_ref[...] = (
            jnp.dot(z, w_ref[...], preferred_element_type=jnp.float32) * dis
        )

    row = pl.BlockSpec((bm, d), lambda i: (i, 0))
    return pl.pallas_call(
        body,
        grid=(m // bm,),
        in_specs=[
            row,
            row,
            row,
            pl.BlockSpec((bm, 1), lambda i: (i, 0)),
            pl.BlockSpec((bm, 1), lambda i: (i, 0)),
            pl.BlockSpec((1, d), lambda i: (0, 0)),
            pl.BlockSpec((d, d), lambda i: (0, 0)),
        ],
        out_specs=row,
        out_shape=jax.ShapeDtypeStruct((m, d), jnp.float32),
    )(p0, p1, h1p, d0, d1, b1, w2)


def _tc_layer3(q0, q1, h2p, d0, d1, b2, bm=1000):
    m, d = h2p.shape

    def body(q0_ref, q1_ref, h_ref, d0_ref, d1_ref, b_ref, o_ref):
        dis = lax.rsqrt(d0_ref[...] + d1_ref[...] + 1.0)
        o_ref[...] = dis * (q0_ref[...] + q1_ref[...] + h_ref[...]) + b_ref[...]

    row = pl.BlockSpec((bm, d), lambda i: (i, 0))
    return pl.pallas_call(
        body,
        grid=(m // bm,),
        in_specs=[
            row,
            row,
            row,
            pl.BlockSpec((bm, 1), lambda i: (i, 0)),
            pl.BlockSpec((bm, 1), lambda i: (i, 0)),
            pl.BlockSpec((1, d), lambda i: (0, 0)),
        ],
        out_specs=row,
        out_shape=jax.ShapeDtypeStruct((m, d), jnp.float32),
    )(q0, q1, h2p, d0, d1, b2)


def kernel(x, edge_index, W1, b1, W2, b2):
    n, d = x.shape
    e = edge_index.shape[1]

    nch = _IDXB * (-(-e // (_NW * _CHUNK * _IDXB)))
    epw = nch * _CHUNK
    ep = epw * _NW
    npad = 256 * (-(-(n + 1) // 256))

    src = edge_index[0].astype(jnp.int32)
    dst = edge_index[1].astype(jnp.int32)
    pad = ep - e
    pad_src = jnp.arange(pad, dtype=jnp.int32) % n
    pad_dst = n + jnp.arange(pad, dtype=jnp.int32) % (npad - n)
    src3 = jnp.concatenate([src, pad_src]).reshape(_NW, nch, _CHUNK)
    dst3 = jnp.concatenate([dst, pad_dst]).reshape(_NW, nch, _CHUNK)
    ei4 = jnp.stack([src3, dst3], axis=2)

    rps = npad // _NS
    zeros_1d = jnp.zeros((npad,), jnp.float32)
    zeros_d = jnp.zeros((rps, d), jnp.float32)

    u1 = _tc_matmul(x, W1)
    degp = _sc_degree(dst3, zeros_1d, npad, nch)
    d0 = degp[0, :n].reshape(n, 1)
    d1 = degp[1, :n].reshape(n, 1)

    h1p = _tc_scale(u1, d0, d1)

    p = _sc_edge_agg(h1p, ei4, zeros_d, npad, nch)
    h2p = _tc_layer2(p[0, :n, :], p[1, :n, :], h1p, d0, d1, b1.reshape(1, d), W2)

    q = _sc_edge_agg(h2p, ei4, zeros_d, npad, nch)
    return _tc_layer3(q[0, :n, :], q[1, :n, :], h2p, d0, d1, b2.reshape(1, d))


# speedup vs baseline: 3.2596x; 1.0233x over previous
import functools

import jax
import jax.numpy as jnp
from jax import lax
from jax.experimental import pallas as pl
from jax.experimental.pallas import tpu as pltpu
from jax.experimental.pallas import tpu_sc as plsc

_NC = 2
_NS = 16
_NW = _NC * _NS
_CHUNK = 120


def _sc_mesh():
    return plsc.VectorSubcoreMesh(
        core_axis_name="c", subcore_axis_name="s", num_cores=_NC, num_subcores=_NS
    )


def _sc_degree(dst3, zeros_hbm, npad, nch):
    rps = npad // _NS

    @functools.partial(
        pl.kernel,
        out_type=jax.ShapeDtypeStruct((_NC, npad), jnp.float32),
        mesh=_sc_mesh(),
        scratch_types=[
            pltpu.VMEM((nch, _CHUNK), jnp.int32),
            pltpu.VMEM((_CHUNK,), jnp.float32),
            pltpu.VMEM_SHARED((npad,), jnp.float32),
            pltpu.SemaphoreType.DMA,
        ],
    )
    def k(dst_hbm, zeros_h, ones_h, out_hbm, didx, ones_v, acc, sem):
        c = lax.axis_index("c")
        s = lax.axis_index("s")
        wid = s * _NC + c
        r0 = s * rps
        pltpu.sync_copy(zeros_h.at[pl.ds(r0, rps)], acc.at[pl.ds(r0, rps)])
        pltpu.sync_copy(ones_h, ones_v)
        pltpu.sync_copy(dst_hbm.at[wid], didx)
        plsc.subcore_barrier()

        def body(i, carry):
            pltpu.async_copy(ones_v, acc.at[didx.at[i]], sem, add=True)
            return carry

        lax.fori_loop(0, nch, body, 0)

        def drain(i, carry):
            pltpu.make_async_copy(ones_v, acc.at[pl.ds(0, _CHUNK)], sem).wait()
            return carry

        lax.fori_loop(0, nch, drain, 0)
        plsc.subcore_barrier()
        pltpu.sync_copy(acc.at[pl.ds(r0, rps)], out_hbm.at[c, pl.ds(r0, rps)])

    return k(dst3, zeros_hbm, jnp.ones((_CHUNK,), jnp.float32))


_NBUF = 3
_LOOK = 2
_IDXB = 2 * _NBUF


def _sc_edge_agg(h, ei4, zeros_hbm, npad, nch):
    n, d = h.shape
    rps = npad // _NS
    assert nch % _IDXB == 0

    @functools.partial(
        pl.kernel,
        out_type=jax.ShapeDtypeStruct((_NC, npad, d), jnp.float32),
        mesh=_sc_mesh(),
        scratch_types=[pltpu.VMEM((_CHUNK, d), jnp.float32)] * _NBUF
        + [pltpu.VMEM((2, _CHUNK), jnp.int32)] * _IDXB
        + [pltpu.VMEM_SHARED((npad, d), jnp.float32)]
        + [pltpu.SemaphoreType.DMA] * (2 * _NBUF + _IDXB),
    )
    def k(h_hbm, ei_hbm, zeros_h, out_hbm, *rest):
        rows = rest[:_NBUF]
        idxs = rest[_NBUF : _NBUF + _IDXB]
        acc = rest[_NBUF + _IDXB]
        sems = rest[_NBUF + _IDXB + 1 :]
        gsem = sems[:_NBUF]
        ssem = sems[_NBUF : 2 * _NBUF]
        isem = sems[2 * _NBUF :]
        c = lax.axis_index("c")
        s = lax.axis_index("s")
        wid = s * _NC + c
        r0 = s * rps
        pltpu.sync_copy(zeros_h, acc.at[pl.ds(r0, rps)])

        def idx_start(chunk, q):
            pltpu.async_copy(ei_hbm.at[wid, chunk], idxs[q], isem[q])

        def idx_wait(q):
            pltpu.make_async_copy(ei_hbm.at[0, 0], idxs[q], isem[q]).wait()

        def gather_start(q, b):
            pltpu.async_copy(h_hbm.at[idxs[q].at[0]], rows[b], gsem[b])

        def gather_wait(b):
            pltpu.make_async_copy(h_hbm.at[pl.ds(0, _CHUNK)], rows[b], gsem[b]).wait()

        def scatter_start(q, b):
            pltpu.async_copy(rows[b], acc.at[idxs[q].at[1]], ssem[b], add=True)

        def scatter_wait(b):
            pltpu.make_async_copy(rows[b], acc.at[pl.ds(0, _CHUNK)], ssem[b]).wait()

        for j in range(_NBUF):
            idx_start(j, j)
        plsc.subcore_barrier()

        def emit_iter(i, u, first):
            b = u % _NBUF
            q = u % _IDXB
            idx_wait(q)
            if not first or u >= _NBUF:
                scatter_wait(b)
            gather_start(q, b)
            idx_start(lax.rem(i + _NBUF, nch), (u + _NBUF) % _IDXB)
            if not first or u >= _LOOK:
                bl = (u - _LOOK) % _NBUF
                gather_wait(bl)
                scatter_start((u - _LOOK) % _IDXB, bl)

        for u in range(_IDXB):
            emit_iter(jnp.int32(u), u, True)

        def blk(t, carry):
            for u in range(_IDXB):
                emit_iter(t * _IDXB + u, u, False)
            return carry

        lax.fori_loop(1, nch // _IDXB, blk, 0)
        for j in range(_LOOK):
            u = nch + j
            bl = (u - _LOOK) % _NBUF
            gather_wait(bl)
            scatter_start((u - _LOOK) % _IDXB, bl)
        for b in range(_NBUF):
            scatter_wait(b)
        for j in range(_NBUF):
            idx_wait((nch + j) % _IDXB)
        plsc.subcore_barrier()
        pltpu.sync_copy(acc.at[pl.ds(r0, rps)], out_hbm.at[c, pl.ds(r0, rps)])

    return k(h, ei4, zeros_hbm)


def _tc_matmul(x, w1, bm=1000):
    m, d = x.shape

    def body(x_ref, w_ref, o_ref):
        o_ref[...] = jnp.dot(
            x_ref[...], w_ref[...], preferred_element_type=jnp.float32
        )

    return pl.pallas_call(
        body,
        grid=(m // bm,),
        in_specs=[
            pl.BlockSpec((bm, d), lambda i: (i, 0)),
            pl.BlockSpec((d, d), lambda i: (0, 0)),
        ],
        out_specs=pl.BlockSpec((bm, d), lambda i: (i, 0)),
        out_shape=jax.ShapeDtypeStruct((m, d), jnp.float32),
    )(x, w1)


def _tc_layer1f(x, w1, d0, d1, bm=1000):
    m, d = x.shape

    def body(x_ref, w_ref, d0_ref, d1_ref, o_ref):
        dis = lax.rsqrt(d0_ref[...] + d1_ref[...] + 1.0)
        o_ref[...] = (
            jnp.dot(x_ref[...], w_ref[...], preferred_element_type=jnp.float32)
            * dis
        )

    return pl.pallas_call(
        body,
        grid=(m // bm,),
        in_specs=[
            pl.BlockSpec((bm, d), lambda i: (i, 0)),
            pl.BlockSpec((d, d), lambda i: (0, 0)),
            pl.BlockSpec((bm, 1), lambda i: (i, 0)),
            pl.BlockSpec((bm, 1), lambda i: (i, 0)),
        ],
        out_specs=pl.BlockSpec((bm, d), lambda i: (i, 0)),
        out_shape=jax.ShapeDtypeStruct((m, d), jnp.float32),
    )(x, w1, d0, d1)


def _tc_scale(u, d0, d1, bm=1000):
    m, d = u.shape

    def body(u_ref, d0_ref, d1_ref, o_ref):
        dis = lax.rsqrt(d0_ref[...] + d1_ref[...] + 1.0)
        o_ref[...] = u_ref[...] * dis

    return pl.pallas_call(
        body,
        grid=(m // bm,),
        in_specs=[
            pl.BlockSpec((bm, d), lambda i: (i, 0)),
            pl.BlockSpec((bm, 1), lambda i: (i, 0)),
            pl.BlockSpec((bm, 1), lambda i: (i, 0)),
        ],
        out_specs=pl.BlockSpec((bm, d), lambda i: (i, 0)),
        out_shape=jax.ShapeDtypeStruct((m, d), jnp.float32),
    )(u, d0, d1)


def _tc_layer2(p0, p1, h1p, d0, d1, b1, w2, bm=1000):
    m, d = h1p.shape

    def body(p0_ref, p1_ref, h_ref, d0_ref, d1_ref, b_ref, w_ref, o_ref):
        dis = lax.rsqrt(d0_ref[...] + d1_ref[...] + 1.0)
        z = dis * (p0_ref[...] + p1_ref[...] + h_ref[...]) + b_ref[...]
        z = jnp.maximum(z, 0.0)
        o_ref[...] = (
            jnp.dot(z, w_ref[...], preferred_element_type=jnp.float32) * dis
        )

    row = pl.BlockSpec((bm, d), lambda i: (i, 0))
    return pl.pallas_call(
        body,
        grid=(m // bm,),
        in_specs=[
            row,
            row,
            row,
            pl.BlockSpec((bm, 1), lambda i: (i, 0)),
            pl.BlockSpec((bm, 1), lambda i: (i, 0)),
            pl.BlockSpec((1, d), lambda i: (0, 0)),
            pl.BlockSpec((d, d), lambda i: (0, 0)),
        ],
        out_specs=row,
        out_shape=jax.ShapeDtypeStruct((m, d), jnp.float32),
    )(p0, p1, h1p, d0, d1, b1, w2)


def _tc_layer3(q0, q1, h2p, d0, d1, b2, bm=1000):
    m, d = h2p.shape

    def body(q0_ref, q1_ref, h_ref, d0_ref, d1_ref, b_ref, o_ref):
        dis = lax.rsqrt(d0_ref[...] + d1_ref[...] + 1.0)
        o_ref[...] = dis * (q0_ref[...] + q1_ref[...] + h_ref[...]) + b_ref[...]

    row = pl.BlockSpec((bm, d), lambda i: (i, 0))
    return pl.pallas_call(
        body,
        grid=(m // bm,),
        in_specs=[
            row,
            row,
            row,
            pl.BlockSpec((bm, 1), lambda i: (i, 0)),
            pl.BlockSpec((bm, 1), lambda i: (i, 0)),
            pl.BlockSpec((1, d), lambda i: (0, 0)),
        ],
        out_specs=row,
        out_shape=jax.ShapeDtypeStruct((m, d), jnp.float32),
    )(q0, q1, h2p, d0, d1, b2)


def kernel(x, edge_index, W1, b1, W2, b2):
    n, d = x.shape
    e = edge_index.shape[1]

    nch = _IDXB * (-(-e // (_NW * _CHUNK * _IDXB)))
    epw = nch * _CHUNK
    ep = epw * _NW
    npad = 256 * (-(-(n + 1) // 256))

    src = edge_index[0].astype(jnp.int32)
    dst = edge_index[1].astype(jnp.int32)
    pad = ep - e
    pad_src = jnp.arange(pad, dtype=jnp.int32) % n
    pad_dst = n + jnp.arange(pad, dtype=jnp.int32) % (npad - n)
    src3 = jnp.concatenate([src, pad_src]).reshape(_NW, nch, _CHUNK)
    dst3 = jnp.concatenate([dst, pad_dst]).reshape(_NW, nch, _CHUNK)
    ei4 = jnp.stack([src3, dst3], axis=2)

    rps = npad // _NS
    zeros_1d = jnp.zeros((npad,), jnp.float32)
    zeros_d = jnp.zeros((rps, d), jnp.float32)

    degp = _sc_degree(dst3, zeros_1d, npad, nch)
    d0 = degp[0, :n].reshape(n, 1)
    d1 = degp[1, :n].reshape(n, 1)

    h1p = _tc_layer1f(x, W1, d0, d1)

    p = _sc_edge_agg(h1p, ei4, zeros_d, npad, nch)
    h2p = _tc_layer2(p[0, :n, :], p[1, :n, :], h1p, d0, d1, b1.reshape(1, d), W2)

    q = _sc_edge_agg(h2p, ei4, zeros_d, npad, nch)
    return _tc_layer3(q[0, :n, :], q[1, :n, :], h2p, d0, d1, b2.reshape(1, d))


# speedup vs baseline: 3.2654x; 1.0018x over previous
import functools

import jax
import jax.numpy as jnp
from jax import lax
from jax.experimental import pallas as pl
from jax.experimental.pallas import tpu as pltpu
from jax.experimental.pallas import tpu_sc as plsc

_NC = 2
_NS = 16
_NW = _NC * _NS
_CHUNK = 120


def _sc_mesh():
    return plsc.VectorSubcoreMesh(
        core_axis_name="c", subcore_axis_name="s", num_cores=_NC, num_subcores=_NS
    )


def _sc_degree(dst3, zeros_hbm, npad, nch):
    rps = npad // _NS

    @functools.partial(
        pl.kernel,
        out_type=jax.ShapeDtypeStruct((_NC, npad), jnp.float32),
        mesh=_sc_mesh(),
        scratch_types=[
            pltpu.VMEM((nch, _CHUNK), jnp.int32),
            pltpu.VMEM((_CHUNK,), jnp.float32),
            pltpu.VMEM_SHARED((npad,), jnp.float32),
            pltpu.SemaphoreType.DMA,
        ],
    )
    def k(dst_hbm, zeros_h, ones_h, out_hbm, didx, ones_v, acc, sem):
        c = lax.axis_index("c")
        s = lax.axis_index("s")
        wid = s * _NC + c
        r0 = s * rps
        pltpu.sync_copy(zeros_h.at[pl.ds(r0, rps)], acc.at[pl.ds(r0, rps)])
        pltpu.sync_copy(ones_h, ones_v)
        pltpu.sync_copy(dst_hbm.at[wid], didx)
        plsc.subcore_barrier()

        def body(i, carry):
            pltpu.async_copy(ones_v, acc.at[didx.at[i]], sem, add=True)
            return carry

        lax.fori_loop(0, nch, body, 0)

        def drain(i, carry):
            pltpu.make_async_copy(ones_v, acc.at[pl.ds(0, _CHUNK)], sem).wait()
            return carry

        lax.fori_loop(0, nch, drain, 0)
        plsc.subcore_barrier()
        pltpu.sync_copy(acc.at[pl.ds(r0, rps)], out_hbm.at[c, pl.ds(r0, rps)])

    return k(dst3, zeros_hbm, jnp.ones((_CHUNK,), jnp.float32))


_NBUF = 3
_LOOK = 2
_IDXB = 2 * _NBUF


def _sc_edge_agg(h, ei4, zeros_hbm, npad, nch):
    n, d = h.shape
    rps = npad // _NS
    assert nch % _IDXB == 0

    @functools.partial(
        pl.kernel,
        out_type=jax.ShapeDtypeStruct((_NC, npad, d), jnp.float32),
        mesh=_sc_mesh(),
        scratch_types=[pltpu.VMEM((_CHUNK, d), jnp.float32)] * _NBUF
        + [pltpu.VMEM((2, _CHUNK), jnp.int32)] * _IDXB
        + [pltpu.VMEM_SHARED((npad, d), jnp.float32)]
        + [pltpu.SemaphoreType.DMA] * (2 * _NBUF + _IDXB),
    )
    def k(h_hbm, ei_hbm, zeros_h, out_hbm, *rest):
        rows = rest[:_NBUF]
        idxs = rest[_NBUF : _NBUF + _IDXB]
        acc = rest[_NBUF + _IDXB]
        sems = rest[_NBUF + _IDXB + 1 :]
        gsem = sems[:_NBUF]
        ssem = sems[_NBUF : 2 * _NBUF]
        isem = sems[2 * _NBUF :]
        c = lax.axis_index("c")
        s = lax.axis_index("s")
        wid = s * _NC + c
        r0 = s * rps
        pltpu.sync_copy(zeros_h, acc.at[pl.ds(r0, rps)])

        def idx_start(chunk, q):
            pltpu.async_copy(ei_hbm.at[wid, chunk], idxs[q], isem[q])

        def idx_wait(q):
            pltpu.make_async_copy(ei_hbm.at[0, 0], idxs[q], isem[q]).wait()

        def gather_start(q, b):
            pltpu.async_copy(h_hbm.at[idxs[q].at[0]], rows[b], gsem[b])

        def gather_wait(b):
            pltpu.make_async_copy(h_hbm.at[pl.ds(0, _CHUNK)], rows[b], gsem[b]).wait()

        def scatter_start(q, b):
            pltpu.async_copy(rows[b], acc.at[idxs[q].at[1]], ssem[b], add=True)

        def scatter_wait(b):
            pltpu.make_async_copy(rows[b], acc.at[pl.ds(0, _CHUNK)], ssem[b]).wait()

        for j in range(_NBUF):
            idx_start(j, j)
        plsc.subcore_barrier()

        def emit_iter(i, u, first):
            b = u % _NBUF
            q = u % _IDXB
            idx_wait(q)
            if not first or u >= _NBUF:
                scatter_wait(b)
            gather_start(q, b)
            idx_start(lax.rem(i + _NBUF, nch), (u + _NBUF) % _IDXB)
            if not first or u >= _LOOK:
                bl = (u - _LOOK) % _NBUF
                gather_wait(bl)
                scatter_start((u - _LOOK) % _IDXB, bl)

        for u in range(_IDXB):
            emit_iter(jnp.int32(u), u, True)

        def blk(t, carry):
            for u in range(_IDXB):
                emit_iter(t * _IDXB + u, u, False)
            return carry

        lax.fori_loop(1, nch // _IDXB, blk, 0)
        for j in range(_LOOK):
            u = nch + j
            bl = (u - _LOOK) % _NBUF
            gather_wait(bl)
            scatter_start((u - _LOOK) % _IDXB, bl)
        for b in range(_NBUF):
            scatter_wait(b)
        for j in range(_NBUF):
            idx_wait((nch + j) % _IDXB)
        plsc.subcore_barrier()
        pltpu.sync_copy(acc.at[pl.ds(r0, rps)], out_hbm.at[c, pl.ds(r0, rps)])

    return k(h, ei4, zeros_hbm)


def _tc_layer1f(x, w1, d0, d1, bm=1000):
    m, d = x.shape

    def body(x_ref, w_ref, d0_ref, d1_ref, o_ref):
        dis = lax.rsqrt(d0_ref[...] + d1_ref[...] + 1.0)
        o_ref[...] = (
            jnp.dot(x_ref[...], w_ref[...], preferred_element_type=jnp.float32)
            * dis
        )

    return pl.pallas_call(
        body,
        grid=(m // bm,),
        in_specs=[
            pl.BlockSpec((bm, d), lambda i: (i, 0)),
            pl.BlockSpec((d, d), lambda i: (0, 0)),
            pl.BlockSpec((bm, 1), lambda i: (i, 0)),
            pl.BlockSpec((bm, 1), lambda i: (i, 0)),
        ],
        out_specs=pl.BlockSpec((bm, d), lambda i: (i, 0)),
        out_shape=jax.ShapeDtypeStruct((m, d), jnp.float32),
    )(x, w1, d0, d1)


def _tc_layer2(p0, p1, h1p, d0, d1, b1, w2, bm=1000):
    m, d = h1p.shape

    def body(p0_ref, p1_ref, h_ref, d0_ref, d1_ref, b_ref, w_ref, o_ref):
        dis = lax.rsqrt(d0_ref[...] + d1_ref[...] + 1.0)
        z = dis * (p0_ref[...] + p1_ref[...] + h_ref[...]) + b_ref[...]
        z = jnp.maximum(z, 0.0)
        o_ref[...] = (
            jnp.dot(z, w_ref[...], preferred_element_type=jnp.float32) * dis
        )

    row = pl.BlockSpec((bm, d), lambda i: (i, 0))
    return pl.pallas_call(
        body,
        grid=(m // bm,),
        in_specs=[
            row,
            row,
            row,
            pl.BlockSpec((bm, 1), lambda i: (i, 0)),
            pl.BlockSpec((bm, 1), lambda i: (i, 0)),
            pl.BlockSpec((1, d), lambda i: (0, 0)),
            pl.BlockSpec((d, d), lambda i: (0, 0)),
        ],
        out_specs=row,
        out_shape=jax.ShapeDtypeStruct((m, d), jnp.float32),
    )(p0, p1, h1p, d0, d1, b1, w2)


def _tc_layer3(q0, q1, h2p, d0, d1, b2, bm=1000):
    m, d = h2p.shape

    def body(q0_ref, q1_ref, h_ref, d0_ref, d1_ref, b_ref, o_ref):
        dis = lax.rsqrt(d0_ref[...] + d1_ref[...] + 1.0)
        o_ref[...] = dis * (q0_ref[...] + q1_ref[...] + h_ref[...]) + b_ref[...]

    row = pl.BlockSpec((bm, d), lambda i: (i, 0))
    return pl.pallas_call(
        body,
        grid=(m // bm,),
        in_specs=[
            row,
            row,
            row,
            pl.BlockSpec((bm, 1), lambda i: (i, 0)),
            pl.BlockSpec((bm, 1), lambda i: (i, 0)),
            pl.BlockSpec((1, d), lambda i: (0, 0)),
        ],
        out_specs=row,
        out_shape=jax.ShapeDtypeStruct((m, d), jnp.float32),
    )(q0, q1, h2p, d0, d1, b2)


def kernel(x, edge_index, W1, b1, W2, b2):
    n, d = x.shape
    e = edge_index.shape[1]

    nch = _IDXB * (-(-e // (_NW * _CHUNK * _IDXB)))
    epw = nch * _CHUNK
    ep = epw * _NW
    npad = 256 * (-(-(n + 1) // 256))

    src = edge_index[0].astype(jnp.int32)
    dst = edge_index[1].astype(jnp.int32)
    pad = ep - e
    pad_src = jnp.arange(pad, dtype=jnp.int32) % n
    pad_dst = n + jnp.arange(pad, dtype=jnp.int32) % (npad - n)
    src3 = jnp.concatenate([src, pad_src]).reshape(_NW, nch, _CHUNK)
    dst3 = jnp.concatenate([dst, pad_dst]).reshape(_NW, nch, _CHUNK)
    ei4 = jnp.stack([src3, dst3], axis=2)

    rps = npad // _NS
    zeros_1d = jnp.zeros((npad,), jnp.float32)
    zeros_d = jnp.zeros((rps, d), jnp.float32)

    degp = _sc_degree(dst3, zeros_1d, npad, nch)
    d0 = degp[0, :n].reshape(n, 1)
    d1 = degp[1, :n].reshape(n, 1)

    h1p = _tc_layer1f(x, W1, d0, d1)

    p = _sc_edge_agg(h1p, ei4, zeros_d, npad, nch)
    h2p = _tc_layer2(p[0, :n, :], p[1, :n, :], h1p, d0, d1, b1.reshape(1, d), W2)

    q = _sc_edge_agg(h2p, ei4, zeros_d, npad, nch)
    return _tc_layer3(q[0, :n, :], q[1, :n, :], h2p, d0, d1, b2.reshape(1, d))
